# Initial kernel scaffold; baseline (speedup 1.0000x reference)
import jax
import jax.numpy as jnp
from jax.experimental import pallas as pl


def kernel(x, depth, B_c, exp_negative_beta_b):
    raise NotImplementedError("write your pallas kernel here")


# speedup vs baseline: 17.5413x; 17.5413x over previous
import jax
import jax.numpy as jnp
from jax import lax
from jax.experimental import pallas as pl

_GROUPS = 10
_K = 500


def _lane_scalar(vec, lane_idx, lane_iota):
    return jnp.sum(jnp.where(lane_iota == lane_idx, vec, 0.0))


def _backscatter_body(x_ref, d_ref, bc_ref, enb_ref, o_ref):
    B, C, R, L = x_ref.shape
    N = R * L
    f32 = jnp.float32
    i32 = jnp.int32
    idx_bits = int(N - 1).bit_length()

    dall = d_ref[...]
    dmin = jnp.min(dall)
    dmax = jnp.max(dall)

    def two_sum(a, b):
        s = a + b
        v = s - a
        e = (a - (s - v)) + (b - v)
        return s, e

    def split(a):
        c = a * f32(4097.0)
        hi = c - (c - a)
        return hi, a - hi

    def two_prod(a, b):
        p = a * b
        ah, al = split(a)
        bh, bl = split(b)
        e = ((ah * bh - p) + ah * bl + al * bh) + al * bl
        return p, e

    lane = lax.broadcasted_iota(i32, (1, L), 1)
    g = f32(_GROUPS)
    dh, dl = two_sum(dmax, -dmin)
    q1 = dh / g
    p, pe = two_prod(q1, g)
    t, te = two_sum(dh, -p)
    r = t + ((te - pe) + dl)
    q2 = r / g
    s_hi, s_lo = two_sum(q1, q2)
    idxv = lane.astype(f32)
    ph, pe2 = two_prod(jnp.full((1, L), s_hi), idxv)
    plo = pe2 + s_lo * idxv
    th, te2 = two_sum(ph, jnp.full((1, L), dmin))
    iv = th + (te2 + plo)
    iv = jnp.where(lane == 0, f32(0.0), iv)
    iv = jnp.where(lane == _GROUPS, dmax, iv)
    intervals = [_lane_scalar(iv, j, lane) for j in range(_GROUPS + 1)]

    cnts = [jnp.int32(0) for _ in range(_GROUPS)]
    for b in range(B):
        db = d_ref[b]
        gt = jnp.zeros((R, L), i32)
        for j in range(_GROUPS + 1):
            gt = gt + (db > intervals[j]).astype(i32)
        gmap = gt - 1
        for i in range(_GROUPS):
            cnts[i] = cnts[i] + jnp.sum((gmap == i).astype(i32))
    ks = []
    for i in range(_GROUPS):
        numpix = cnts[i].astype(f32) / f32(B)
        kf = jnp.minimum(jnp.ceil(numpix * f32(0.01)), f32(_K))
        ks.append(kf.astype(i32))

    lgrows = [jnp.log(enb_ref[c : c + 1, :]) for c in range(C)]
    bcrows = [bc_ref[c : c + 1, :] for c in range(C)]

    pix_idx = (
        lax.broadcasted_iota(i32, (R, L), 0) * L
        + lax.broadcasted_iota(i32, (R, L), 1)
    )

    num_acc = f32(0.0)
    den_acc = f32(0.0)

    for b in range(B):
        db = d_ref[b]
        bright = (x_ref[b, 0] + x_ref[b, 1] + x_ref[b, 2]) / f32(C)
        bbits = lax.bitcast_convert_type(bright, i32)
        mbits = lax.bitcast_convert_type(bright * f32(1000.0), i32)
        gt = jnp.zeros((R, L), i32)
        for j in range(_GROUPS + 1):
            gt = gt + (db > intervals[j]).astype(i32)
        gmap = gt - 1
        v = [
            jnp.where(gmap == i, bbits, mbits) for i in range(_GROUPS)
        ]

        def p1_body(it, ts):
            bitval = jnp.left_shift(jnp.int32(1), 30 - it)
            new = []
            for i in range(_GROUPS):
                cand = ts[i] + bitval
                cnt = jnp.sum((v[i] < cand).astype(i32))
                new.append(jnp.where(cnt < ks[i], cand, ts[i]))
            return tuple(new)

        ts = lax.fori_loop(
            0, 31, p1_body, tuple(jnp.int32(0) for _ in range(_GROUPS))
        )

        c1 = [jnp.sum((v[i] < ts[i]).astype(i32)) for i in range(_GROUPS)]
        tie = [v[i] == ts[i] for i in range(_GROUPS)]

        def p2_body(it, js):
            bitval = jnp.left_shift(jnp.int32(1), idx_bits - 1 - it)
            new = []
            for i in range(_GROUPS):
                cand = js[i] + bitval
                cnt2 = jnp.sum((tie[i] & (pix_idx < cand)).astype(i32))
                new.append(jnp.where(c1[i] + cnt2 < ks[i], cand, js[i]))
            return tuple(new)

        js = lax.fori_loop(
            0, idx_bits, p2_body, tuple(jnp.int32(0) for _ in range(_GROUPS))
        )

        sel = jnp.zeros((R, L), jnp.bool_)
        for i in range(_GROUPS):
            si = (v[i] < ts[i]) | (tie[i] & (pix_idx <= js[i]))
            si = si & (ks[i] > 0)
            sel = sel | si

        rsum = jnp.zeros((R, L), f32)
        for c in range(C):
            tgt = bcrows[c] * (f32(1.0) - jnp.exp(db * lgrows[c]))
            rsum = rsum + jnp.abs(x_ref[b, c] - tgt)
        num_acc = num_acc + jnp.sum(jnp.where(sel, rsum, f32(0.0)))
        den_acc = den_acc + jnp.sum(sel.astype(f32))

    o_ref[...] = (num_acc / den_acc) * jnp.ones((1, 1), f32)


def kernel(x, depth, B_c, exp_negative_beta_b):
    B, C, H, W = x.shape
    N = H * W
    L = 128
    R = N // L
    xr = x.reshape(B, C, R, L)
    dr = depth.reshape(B, R, L)
    bc = jnp.zeros((8, L), jnp.float32).at[:C].set(
        jnp.broadcast_to(B_c.reshape(C, 1), (C, L))
    )
    enb = jnp.ones((8, L), jnp.float32).at[:C].set(
        jnp.broadcast_to(exp_negative_beta_b.reshape(C, 1), (C, L))
    )
    out = pl.pallas_call(
        _backscatter_body,
        out_shape=jax.ShapeDtypeStruct((1, 1), jnp.float32),
    )(xr, dr, bc, enb)
    return out[0, 0]


# speedup vs baseline: 24.1257x; 1.3754x over previous
import jax
import jax.numpy as jnp
from jax import lax
from jax.experimental import pallas as pl

_GROUPS = 10
_K = 500


def _lane_scalar(vec, lane_idx, lane_iota):
    return jnp.sum(jnp.where(lane_iota == lane_idx, vec, 0.0))


def _backscatter_body(x_ref, d_ref, bc_ref, enb_ref, o_ref):
    B, C, R, L = x_ref.shape
    N = R * L
    f32 = jnp.float32
    i32 = jnp.int32
    idx_bits = int(N - 1).bit_length()

    dall = d_ref[...]
    dmin = jnp.min(dall)
    dmax = jnp.max(dall)

    def two_sum(a, b):
        s = a + b
        v = s - a
        e = (a - (s - v)) + (b - v)
        return s, e

    def split(a):
        c = a * f32(4097.0)
        hi = c - (c - a)
        return hi, a - hi

    def two_prod(a, b):
        p = a * b
        ah, al = split(a)
        bh, bl = split(b)
        e = ((ah * bh - p) + ah * bl + al * bh) + al * bl
        return p, e

    lane = lax.broadcasted_iota(i32, (1, L), 1)
    g = f32(_GROUPS)
    dh, dl = two_sum(dmax, -dmin)
    q1 = dh / g
    p, pe = two_prod(q1, g)
    t, te = two_sum(dh, -p)
    r = t + ((te - pe) + dl)
    q2 = r / g
    s_hi, s_lo = two_sum(q1, q2)
    idxv = lane.astype(f32)
    ph, pe2 = two_prod(jnp.full((1, L), s_hi), idxv)
    plo = pe2 + s_lo * idxv
    th, te2 = two_sum(ph, jnp.full((1, L), dmin))
    iv = th + (te2 + plo)
    iv = jnp.where(lane == 0, f32(0.0), iv)
    iv = jnp.where(lane == _GROUPS, dmax, iv)
    intervals = [_lane_scalar(iv, j, lane) for j in range(_GROUPS + 1)]

    cnts = [jnp.int32(0) for _ in range(_GROUPS)]
    gmaps = []
    for b in range(B):
        db = d_ref[b]
        gt = jnp.zeros((R, L), i32)
        for j in range(_GROUPS + 1):
            gt = gt + (db > intervals[j]).astype(i32)
        gmap = gt - 1
        gmaps.append(gmap)
        for i in range(_GROUPS):
            cnts[i] = cnts[i] + jnp.sum((gmap == i).astype(i32))
    ks = []
    for i in range(_GROUPS):
        numpix = cnts[i].astype(f32) / f32(B)
        kf = jnp.minimum(jnp.ceil(numpix * f32(0.01)), f32(_K))
        ks.append(kf.astype(i32))

    lgrows = [jnp.log(enb_ref[c : c + 1, :]) for c in range(C)]
    bcrows = [bc_ref[c : c + 1, :] for c in range(C)]

    pix_idx = (
        lax.broadcasted_iota(i32, (R, L), 0) * L
        + lax.broadcasted_iota(i32, (R, L), 1)
    )

    num_acc = f32(0.0)
    den_acc = f32(0.0)

    for b in range(B):
        db = d_ref[b]
        bright = (x_ref[b, 0] + x_ref[b, 1] + x_ref[b, 2]) / f32(C)
        bbits = lax.bitcast_convert_type(bright, i32)
        mbits = lax.bitcast_convert_type(bright * f32(1000.0), i32)
        gmap = gmaps[b]
        v = [
            jnp.where(gmap == i, bbits, mbits) for i in range(_GROUPS)
        ]

        def p1_body(it, ts):
            bitval = jnp.left_shift(jnp.int32(1), 30 - it)
            new = []
            for i in range(_GROUPS):
                cand = ts[i] + bitval
                cnt = jnp.sum((v[i] < cand).astype(i32))
                new.append(jnp.where(cnt < ks[i], cand, ts[i]))
            return tuple(new)

        ts = lax.fori_loop(
            0, 31, p1_body, tuple(jnp.int32(0) for _ in range(_GROUPS))
        )

        c1 = [jnp.sum((v[i] < ts[i]).astype(i32)) for i in range(_GROUPS)]
        tie = [v[i] == ts[i] for i in range(_GROUPS)]

        need_multi = jnp.bool_(False)
        for i in range(_GROUPS):
            need_multi = need_multi | ((ks[i] - c1[i] >= 2) & (ks[i] > 0))

        def p2_easy(_):
            return tuple(
                jnp.min(jnp.where(tie[i], pix_idx, jnp.int32(1 << 30)))
                for i in range(_GROUPS)
            )

        def p2_hard(_):
            def p2_body(it, js):
                bitval = jnp.left_shift(jnp.int32(1), idx_bits - 1 - it)
                new = []
                for i in range(_GROUPS):
                    cand = js[i] + bitval
                    cnt2 = jnp.sum((tie[i] & (pix_idx < cand)).astype(i32))
                    new.append(jnp.where(c1[i] + cnt2 < ks[i], cand, js[i]))
                return tuple(new)

            return lax.fori_loop(
                0, idx_bits, p2_body,
                tuple(jnp.int32(0) for _ in range(_GROUPS)),
            )

        js = lax.cond(need_multi, p2_hard, p2_easy, None)

        sel = jnp.zeros((R, L), jnp.bool_)
        for i in range(_GROUPS):
            si = (v[i] < ts[i]) | (tie[i] & (pix_idx <= js[i]))
            si = si & (ks[i] > 0)
            sel = sel | si

        rsum = jnp.zeros((R, L), f32)
        for c in range(C):
            tgt = bcrows[c] * (f32(1.0) - jnp.exp(db * lgrows[c]))
            rsum = rsum + jnp.abs(x_ref[b, c] - tgt)
        num_acc = num_acc + jnp.sum(jnp.where(sel, rsum, f32(0.0)))
        den_acc = den_acc + jnp.sum(sel.astype(f32))

    o_ref[...] = (num_acc / den_acc) * jnp.ones((1, 1), f32)


def kernel(x, depth, B_c, exp_negative_beta_b):
    B, C, H, W = x.shape
    N = H * W
    L = 128
    R = N // L
    xr = x.reshape(B, C, R, L)
    dr = depth.reshape(B, R, L)
    bc = jnp.zeros((8, L), jnp.float32).at[:C].set(
        jnp.broadcast_to(B_c.reshape(C, 1), (C, L))
    )
    enb = jnp.ones((8, L), jnp.float32).at[:C].set(
        jnp.broadcast_to(exp_negative_beta_b.reshape(C, 1), (C, L))
    )
    out = pl.pallas_call(
        _backscatter_body,
        out_shape=jax.ShapeDtypeStruct((1, 1), jnp.float32),
    )(xr, dr, bc, enb)
    return out[0, 0]
